# baseline (speedup 1.0000x reference)
import functools

import jax
import jax.numpy as jnp
import numpy as np
from jax import lax
from jax.experimental import pallas as pl
from jax.experimental.pallas import tpu as pltpu
from jax.experimental.pallas import tpu_sc as plsc

_F = 26
_D = 32
_B = 4096
_VOCAB = 100000
_N = _B * _F
_NC = 2
_NS = 16
_NW = _NC * _NS
_BPW = _N // _NW
_NV = _BPW // 16

_OFF_NP = ((np.arange(_BPW) % _F) * _VOCAB).astype(np.int32)

_mesh = plsc.VectorSubcoreMesh(core_axis_name="c", subcore_axis_name="s")


@functools.partial(
    pl.kernel,
    mesh=_mesh,
    out_type=jax.ShapeDtypeStruct((_N, _D), jnp.float32),
    scratch_types=[
        pltpu.VMEM((_BPW,), jnp.int32),
        pltpu.VMEM((_BPW,), jnp.int32),
        pltpu.VMEM((_BPW, _D), jnp.float32),
        pltpu.SemaphoreType.DMA,
    ],
    compiler_params=pltpu.CompilerParams(use_tc_tiling_on_sc=False),
)
def _sc_gather(table_hbm, idx_hbm, off_hbm, out_hbm, idx_v, off_v, rows_v, sem):
    wid = lax.axis_index("s") * _NC + lax.axis_index("c")
    base = wid * _BPW
    pltpu.sync_copy(idx_hbm.at[pl.ds(base, _BPW)], idx_v)
    pltpu.sync_copy(off_hbm, off_v)

    def _fuse(i, carry):
        s = pl.ds(i * 16, 16)
        idx_v[s] = idx_v[s] + off_v[s]
        return carry

    lax.fori_loop(0, _NV, _fuse, 0, unroll=8)

    pltpu.async_copy(table_hbm.at[idx_v], rows_v, sem).wait()
    pltpu.sync_copy(rows_v, out_hbm.at[pl.ds(base, _BPW)])


def kernel(inputs, table):
    flat_idx = inputs.reshape(-1)
    off = jnp.asarray(_OFF_NP)
    out = _sc_gather(table, flat_idx, off)
    return out.reshape(_B, _F, _D)
